# baseline (speedup 1.0000x reference)
import functools

import jax
import jax.numpy as jnp
from jax import lax
from jax.experimental import pallas as pl
from jax.experimental.pallas import tpu as pltpu
from jax.experimental.pallas import tpu_sc as plsc

_VOCAB = 32000
_HIDDEN = 4096
_BATCH = 2
_SEQ = 2048
_B = _BATCH * _SEQ
_NC = 2
_NS = 16
_NW = _NC * _NS
_BPW = _B // _NW

_SIZES = [16, 8] * 5 + [8]
_OFFS = [sum(_SIZES[:i]) for i in range(len(_SIZES))]
_BUFS = [0 if s == 16 else 1 for s in _SIZES]

_mesh = plsc.VectorSubcoreMesh(core_axis_name="c", subcore_axis_name="s")


@functools.partial(
    pl.kernel,
    out_type=jax.ShapeDtypeStruct((_B, _HIDDEN), jnp.float32),
    mesh=_mesh,
    scratch_types=[
        pltpu.VMEM((_BPW,), jnp.int32),
        pltpu.VMEM((16, _HIDDEN), jnp.float32),
        pltpu.VMEM((8, _HIDDEN), jnp.float32),
        pltpu.SemaphoreType.DMA((2,)),
        pltpu.SemaphoreType.DMA((2,)),
    ],
)
def _embed_gather(idx_hbm, table_hbm, out_hbm, idx_v, buf_a, buf_b,
                  gsem, ssem):
    wid = lax.axis_index("s") * _NC + lax.axis_index("c")
    base = wid * _BPW
    pltpu.sync_copy(idx_hbm.at[pl.ds(base, _BPW)], idx_v)
    bufs = (buf_a, buf_b)

    def g_copy(i):
        b = _BUFS[i]
        return pltpu.make_async_copy(
            table_hbm.at[idx_v.at[pl.ds(_OFFS[i], _SIZES[i])]],
            bufs[b], gsem.at[b])

    def s_copy(i):
        b = _BUFS[i]
        return pltpu.make_async_copy(
            bufs[b], out_hbm.at[pl.ds(base + _OFFS[i], _SIZES[i])],
            ssem.at[b])

    n = len(_SIZES)

    def next_use(i):
        for j in range(i + 1, n):
            if _BUFS[j] == _BUFS[i]:
                return j
        return None

    g_copy(0).start()
    g_copy(1).start()
    store_waited = [False] * n
    for i in range(n):
        g_copy(i).wait()
        s_copy(i).start()
        nxt = next_use(i)
        if nxt is not None:
            s_copy(i).wait()
            store_waited[i] = True
            g_copy(nxt).start()
    for i in range(n):
        if not store_waited[i]:
            s_copy(i).wait()


def kernel(inputs, embed_weight):
    input_ids = inputs[..., 0].reshape(_B)
    attention_mask = inputs[..., 1]
    out = _embed_gather(input_ids, embed_weight)
    return out.reshape(_BATCH, _SEQ, _HIDDEN), attention_mask


# speedup vs baseline: 1.0489x; 1.0489x over previous
import functools

import jax
import jax.numpy as jnp
from jax import lax
from jax.experimental import pallas as pl
from jax.experimental.pallas import tpu as pltpu
from jax.experimental.pallas import tpu_sc as plsc

_VOCAB = 32000
_HIDDEN = 4096
_BATCH = 2
_SEQ = 2048
_B = _BATCH * _SEQ
_NC = 2
_NS = 16
_NW = _NC * _NS
_BPW = _B // _NW
_CHUNK = 8
_NBUF = 3
_NSTEP = _BPW // _CHUNK
_G = (_NSTEP - _NBUF) // _NBUF

_mesh = plsc.VectorSubcoreMesh(core_axis_name="c", subcore_axis_name="s")


@functools.partial(
    pl.kernel,
    out_type=jax.ShapeDtypeStruct((_B, _HIDDEN), jnp.float32),
    mesh=_mesh,
    scratch_types=[
        pltpu.VMEM((_BPW,), jnp.int32),
        pltpu.VMEM((_NBUF, _CHUNK, _HIDDEN), jnp.float32),
        pltpu.SemaphoreType.DMA((_NBUF,)),
        pltpu.SemaphoreType.DMA((_NBUF,)),
    ],
)
def _embed_gather(idx_hbm, table_hbm, out_hbm, idx_v, rows_v, gsem, ssem):
    wid = lax.axis_index("s") * _NC + lax.axis_index("c")
    base = wid * _BPW
    pltpu.sync_copy(idx_hbm.at[pl.ds(base, _BPW)], idx_v)

    def g_copy(c, b):
        return pltpu.make_async_copy(
            table_hbm.at[idx_v.at[pl.ds(c * _CHUNK, _CHUNK)]],
            rows_v.at[b], gsem.at[b])

    def s_copy(c, b):
        return pltpu.make_async_copy(
            rows_v.at[b], out_hbm.at[pl.ds(base + c * _CHUNK, _CHUNK)],
            ssem.at[b])

    for b in range(_NBUF):
        g_copy(b, b).start()

    def outer(g, _):
        for b in range(_NBUF):
            c = g * _NBUF + b
            g_copy(c, b).wait()
            s_copy(c, b).start()
            s_copy(c, b).wait()
            g_copy(c + _NBUF, b).start()
        return ()

    lax.fori_loop(0, _G, outer, ())

    done = _G * _NBUF
    pending = list(range(done, done + _NBUF))
    unissued = list(range(done + _NBUF, _NSTEP))
    waited = []
    while pending:
        c = pending.pop(0)
        b = c % _NBUF
        g_copy(c, b).wait()
        s_copy(c, b).start()
        if unissued:
            nxt = unissued.pop(0)
            s_copy(c, b).wait()
            waited.append(c)
            g_copy(nxt, nxt % _NBUF).start()
            pending.append(nxt)
    for c in range(done, _NSTEP):
        if c not in waited:
            s_copy(c, c % _NBUF).wait()


def kernel(inputs, embed_weight):
    input_ids = inputs[..., 0].reshape(_B)
    attention_mask = inputs[..., 1]
    out = _embed_gather(input_ids, embed_weight)
    return out.reshape(_BATCH, _SEQ, _HIDDEN), attention_mask
